# baseline (speedup 1.0000x reference)
import functools
import math

import jax
import jax.numpy as jnp
from jax import lax
from jax.experimental import pallas as pl
from jax.experimental.pallas import tpu as pltpu
from jax.experimental.pallas import tpu_sc as plsc

B = 16384
FEAT = 64
CHUNK = 128


def _pairs_body(a_ref, out_ref, *, vocab):
    a = a_ref[...]
    lane = (lax.broadcasted_iota(jnp.int32, a.shape, 1)
            + pl.program_id(0) * a.shape[1])
    a = jnp.where(lane < vocab, a, 0.0)
    q4 = a.shape[1] // 4
    a4 = jnp.concatenate(
        [a[:, i * q4:(i + 1) * q4] for i in range(4)], axis=0
    ).astype(jnp.bfloat16)
    n = 4 * FEAT
    ii = lax.broadcasted_iota(jnp.int32, (n, n), 0)
    jj = lax.broadcasted_iota(jnp.int32, (n, n), 1)
    eye = jnp.where(ii == jj, 1.0, 0.0).astype(jnp.bfloat16)
    t = lax.dot_general(a4, eye, (((0,), (0,)), ((), ())),
                        preferred_element_type=jnp.float32)

    def pack(lo, hi):
        ulo = lax.bitcast_convert_type(lo, jnp.uint32) >> 16
        uhi = lax.bitcast_convert_type(hi, jnp.uint32) & jnp.uint32(0xFFFF0000)
        return lax.bitcast_convert_type(ulo | uhi, jnp.float32)

    p01 = pack(t[:, 0:FEAT], t[:, FEAT:2 * FEAT])
    p23 = pack(t[:, 2 * FEAT:3 * FEAT], t[:, 3 * FEAT:4 * FEAT])
    out_ref[...] = jnp.concatenate([p01, p23], axis=1)


def _pairs(tab_t, vocab, lblk):
    grid = ((vocab + lblk - 1) // lblk,)
    rows = grid[0] * (lblk // 4)
    return pl.pallas_call(
        functools.partial(_pairs_body, vocab=vocab),
        grid=grid,
        in_specs=[pl.BlockSpec((FEAT, lblk), lambda i: (0, i))],
        out_specs=pl.BlockSpec((lblk // 4, 2 * FEAT), lambda i: (i, 0)),
        out_shape=jax.ShapeDtypeStruct((rows, 2 * FEAT), jnp.float32),
    )(tab_t)


def _gather3(ids_h, locs_h, ages_h, id_pairs, loc_pairs, age_pairs):
    info = plsc.get_sparse_core_info()
    nw = info.num_cores * info.num_subcores
    b_per_w = B // nw
    n_chunks = b_per_w // CHUNK

    mesh = plsc.VectorSubcoreMesh(core_axis_name="c", subcore_axis_name="s")

    @functools.partial(
        pl.kernel,
        mesh=mesh,
        out_type=[jax.ShapeDtypeStruct((B, 2 * FEAT), jnp.float32)] * 3,
        scratch_types=(
            [pltpu.VMEM((n_chunks, CHUNK), jnp.int32)] * 3
            + [pltpu.VMEM((2, CHUNK, 2 * FEAT), jnp.float32)] * 3
            + [pltpu.SemaphoreType.DMA] * 2
        ),
    )
    def gather_k(ids_r, locs_r, ages_r, idt_h, loct_h, aget_h,
                 out_id, out_loc, out_age,
                 idx0, idx1, idx2, rows0, rows1, rows2, gsem, wsem):
        wid = lax.axis_index("s") * info.num_cores + lax.axis_index("c")
        base = wid * b_per_w
        pltpu.sync_copy(ids_r.at[wid], idx0)
        pltpu.sync_copy(locs_r.at[wid], idx1)
        pltpu.sync_copy(ages_r.at[wid], idx2)
        rows = (rows0, rows1, rows2)
        tabs = (idt_h, loct_h, aget_h)
        idxs = (idx0, idx1, idx2)
        outs = (out_id, out_loc, out_age)
        gathers = [[None] * 3 for _ in range(n_chunks)]
        writes = [[None] * 3 for _ in range(n_chunks)]
        for j in range(n_chunks):
            b = j % 2
            if j >= 2:
                for t in range(3):
                    writes[j - 2][t].wait()
            for t in range(3):
                gathers[j][t] = pltpu.async_copy(
                    tabs[t].at[idxs[t].at[j]], rows[t].at[b], gsem)
            for t in range(3):
                gathers[j][t].wait()
            dst = pl.ds(base + j * CHUNK, CHUNK)
            for t in range(3):
                writes[j][t] = pltpu.async_copy(rows[t].at[b], outs[t].at[dst], wsem)
        for j in range(n_chunks - 2, n_chunks):
            for t in range(3):
                writes[j][t].wait()

    return gather_k(ids_h, locs_h, ages_h, id_pairs, loc_pairs, age_pairs)


_INV_SQRT2 = 1.0 / math.sqrt(2.0)


def _gelu(x):
    return 0.5 * x * (1.0 + lax.erf(x * _INV_SQRT2))


def _ln(x, eps=1e-5):
    mu = jnp.mean(x, axis=-1, keepdims=True)
    var = jnp.mean((x - mu) * (x - mu), axis=-1, keepdims=True)
    return (x - mu) * lax.rsqrt(var + eps)


def _half(buf, flag_ref):
    quarter = flag_ref[...]
    w = jnp.where(quarter >= 2, buf[:, FEAT:2 * FEAT], buf[:, 0:FEAT])
    u = lax.bitcast_convert_type(w, jnp.uint32)
    lo = lax.bitcast_convert_type(u << 16, jnp.float32)
    hi = lax.bitcast_convert_type(u & jnp.uint32(0xFFFF0000), jnp.float32)
    return jnp.where((quarter & 1) != 0, hi, lo)


def _mlp_body(id_ref, loc_ref, age_ref, fid_ref, floc_ref, fage_ref,
              w1_ref, b1_ref, w2_ref, b2_ref, w3_ref, b3_ref, out_ref):
    w1 = w1_ref[...]
    id_emb = _half(id_ref[...], fid_ref)
    loc_emb = _half(loc_ref[...], floc_ref)
    age_emb = _half(age_ref[...], fage_ref)
    h = (
        jnp.dot(id_emb, w1[0:FEAT], preferred_element_type=jnp.float32)
        + jnp.dot(loc_emb, w1[FEAT:2 * FEAT], preferred_element_type=jnp.float32)
        + jnp.dot(age_emb, w1[2 * FEAT:3 * FEAT], preferred_element_type=jnp.float32)
        + b1_ref[...]
    )
    h = _gelu(_ln(h))
    h = jnp.dot(h, w2_ref[...], preferred_element_type=jnp.float32) + b2_ref[...]
    h = _gelu(_ln(h))
    h = jnp.dot(h, w3_ref[...], preferred_element_type=jnp.float32) + b3_ref[...]
    out_ref[...] = _gelu(h)


def _mlp(id_emb, loc_emb, age_emb, fid, floc, fage, W1, b1, W2, b2, W3, b3,
         blk=2048):
    grid = (B // blk,)
    rep = lambda i: (0, 0)
    row = lambda i: (i, 0)
    return pl.pallas_call(
        _mlp_body,
        grid=grid,
        in_specs=[
            pl.BlockSpec((blk, 2 * FEAT), row),
            pl.BlockSpec((blk, 2 * FEAT), row),
            pl.BlockSpec((blk, 2 * FEAT), row),
            pl.BlockSpec((blk, 1), row),
            pl.BlockSpec((blk, 1), row),
            pl.BlockSpec((blk, 1), row),
            pl.BlockSpec((3 * FEAT, 128), rep),
            pl.BlockSpec((1, 128), rep),
            pl.BlockSpec((128, 64), rep),
            pl.BlockSpec((1, 64), rep),
            pl.BlockSpec((64, 128), rep),
            pl.BlockSpec((1, 128), rep),
        ],
        out_specs=pl.BlockSpec((blk, 128), row),
        out_shape=jax.ShapeDtypeStruct((B, 128), jnp.float32),
    )(id_emb, loc_emb, age_emb, fid, floc, fage, W1, b1.reshape(1, -1),
      W2, b2.reshape(1, -1), W3, b3.reshape(1, -1))


def kernel(user_ids, user_locations, user_ages, id_table, loc_table, age_table,
           W1, b1, W2, b2, W3, b3):
    info = plsc.get_sparse_core_info()
    nw = info.num_cores * info.num_subcores
    b_per_w = B // nw
    n_chunks = b_per_w // CHUNK

    ids = user_ids.astype(jnp.int32)
    locs = user_locations.astype(jnp.int32)
    ages = user_ages.astype(jnp.int32)

    def pair_idx(r, lblk):
        q = lax.rem(r, lblk)
        q4 = lblk // 4
        p = (r // lblk) * q4 + lax.rem(q, q4)
        flag = q // q4
        return p, flag

    id_pairs = _pairs(id_table.T, 1000000, 8192)
    loc_pairs = _pairs(loc_table.T, 100000, 8192)
    age_pairs = _pairs(age_table.T, 1000, 1000)

    p_id, f_id = pair_idx(ids, 8192)
    p_loc, f_loc = pair_idx(locs, 8192)
    p_age, f_age = pair_idx(ages, 1000)

    id_emb, loc_emb, age_emb = _gather3(
        p_id.reshape(nw, n_chunks, CHUNK),
        p_loc.reshape(nw, n_chunks, CHUNK),
        p_age.reshape(nw, n_chunks, CHUNK),
        id_pairs, loc_pairs, age_pairs)
    return _mlp(id_emb, loc_emb, age_emb,
                f_id.reshape(B, 1), f_loc.reshape(B, 1), f_age.reshape(B, 1),
                W1, b1, W2, b2, W3, b3)


# speedup vs baseline: 1.0925x; 1.0925x over previous
import functools
import math

import jax
import jax.numpy as jnp
from jax import lax
from jax.experimental import pallas as pl
from jax.experimental.pallas import tpu as pltpu
from jax.experimental.pallas import tpu_sc as plsc

B = 16384
FEAT = 64
CHUNK = 128


def _pairs_body(a_ref, out_ref, *, vocab):
    a = a_ref[...]
    lane = (lax.broadcasted_iota(jnp.int32, a.shape, 1)
            + pl.program_id(0) * a.shape[1])
    a = jnp.where(lane < vocab, a, 0.0)
    p = a.shape[1] // 2
    a2 = jnp.concatenate([a[:, :p], a[:, p:]], axis=0).astype(jnp.bfloat16)
    ii = lax.broadcasted_iota(jnp.int32, (2 * FEAT, 2 * FEAT), 0)
    jj = lax.broadcasted_iota(jnp.int32, (2 * FEAT, 2 * FEAT), 1)
    eye = jnp.where(ii == jj, 1.0, 0.0).astype(jnp.bfloat16)
    out_ref[...] = lax.dot_general(a2, eye, (((0,), (0,)), ((), ())),
                                   preferred_element_type=jnp.float32)


def _pairs(tab_t, vocab, lblk):
    grid = ((vocab + lblk - 1) // lblk,)
    half = grid[0] * (lblk // 2)
    return pl.pallas_call(
        functools.partial(_pairs_body, vocab=vocab),
        grid=grid,
        in_specs=[pl.BlockSpec((FEAT, lblk), lambda i: (0, i))],
        out_specs=pl.BlockSpec((lblk // 2, 2 * FEAT), lambda i: (i, 0)),
        out_shape=jax.ShapeDtypeStruct((half, 2 * FEAT), jnp.float32),
    )(tab_t)


def _gather_n(idx_tabs):
    n = len(idx_tabs)
    info = plsc.get_sparse_core_info()
    nw = info.num_cores * info.num_subcores
    b_per_w = B // nw
    n_chunks = b_per_w // CHUNK

    mesh = plsc.VectorSubcoreMesh(core_axis_name="c", subcore_axis_name="s")

    @functools.partial(
        pl.kernel,
        mesh=mesh,
        out_type=[jax.ShapeDtypeStruct((B, 2 * FEAT), jnp.float32)] * n,
        scratch_types=(
            [pltpu.VMEM((n_chunks, CHUNK), jnp.int32)] * n
            + [pltpu.VMEM((2, CHUNK, 2 * FEAT), jnp.float32)] * n
            + [pltpu.SemaphoreType.DMA] * 2
        ),
    )
    def gather_k(*refs):
        idx_r = refs[:n]
        tabs = refs[n:2 * n]
        outs = refs[2 * n:3 * n]
        idxs = refs[3 * n:4 * n]
        rows = refs[4 * n:5 * n]
        gsem, wsem = refs[5 * n:]
        wid = lax.axis_index("s") * info.num_cores + lax.axis_index("c")
        base = wid * b_per_w
        for t in range(n):
            pltpu.sync_copy(idx_r[t].at[wid], idxs[t])
        gathers = [[None] * n for _ in range(n_chunks)]
        writes = [[None] * n for _ in range(n_chunks)]
        for j in range(n_chunks):
            b = j % 2
            if j >= 2:
                for t in range(n):
                    writes[j - 2][t].wait()
            for t in range(n):
                gathers[j][t] = pltpu.async_copy(
                    tabs[t].at[idxs[t].at[j]], rows[t].at[b], gsem)
            for t in range(n):
                gathers[j][t].wait()
            dst = pl.ds(base + j * CHUNK, CHUNK)
            for t in range(n):
                writes[j][t] = pltpu.async_copy(rows[t].at[b], outs[t].at[dst], wsem)
        for j in range(n_chunks - 2, n_chunks):
            for t in range(n):
                writes[j][t].wait()

    args = [it[0] for it in idx_tabs] + [it[1] for it in idx_tabs]
    res = gather_k(*args)
    return list(res) if isinstance(res, (list, tuple)) else [res]


_INV_SQRT2 = 1.0 / math.sqrt(2.0)


def _gelu(x):
    return 0.5 * x * (1.0 + lax.erf(x * _INV_SQRT2))


def _ln(x, eps=1e-5):
    mu = jnp.mean(x, axis=-1, keepdims=True)
    var = jnp.mean((x - mu) * (x - mu), axis=-1, keepdims=True)
    return (x - mu) * lax.rsqrt(var + eps)


def _half(buf, flag_ref):
    p = flag_ref[...] != 0
    return jnp.where(p, buf[:, FEAT:2 * FEAT], buf[:, 0:FEAT])


def _mlp_body(id_ref, loc_ref, age_ref, fid_ref, floc_ref, fage_ref,
              w1_ref, b1_ref, w2_ref, b2_ref, w3_ref, b3_ref, out_ref):
    w1 = w1_ref[...]
    id_emb = _half(id_ref[...], fid_ref)
    loc_emb = _half(loc_ref[...], floc_ref)
    age_emb = _half(age_ref[...], fage_ref)
    h = (
        jnp.dot(id_emb, w1[0:FEAT], preferred_element_type=jnp.float32)
        + jnp.dot(loc_emb, w1[FEAT:2 * FEAT], preferred_element_type=jnp.float32)
        + jnp.dot(age_emb, w1[2 * FEAT:3 * FEAT], preferred_element_type=jnp.float32)
        + b1_ref[...]
    )
    h = _gelu(_ln(h))
    h = jnp.dot(h, w2_ref[...], preferred_element_type=jnp.float32) + b2_ref[...]
    h = _gelu(_ln(h))
    h = jnp.dot(h, w3_ref[...], preferred_element_type=jnp.float32) + b3_ref[...]
    out_ref[...] = _gelu(h)


def _mlp(id_emb, loc_emb, age_emb, fid, floc, fage, W1, b1, W2, b2, W3, b3,
         blk=2048):
    grid = (B // blk,)
    rep = lambda i: (0, 0)
    row = lambda i: (i, 0)
    return pl.pallas_call(
        _mlp_body,
        grid=grid,
        in_specs=[
            pl.BlockSpec((blk, 2 * FEAT), row),
            pl.BlockSpec((blk, 2 * FEAT), row),
            pl.BlockSpec((blk, 2 * FEAT), row),
            pl.BlockSpec((blk, 1), row),
            pl.BlockSpec((blk, 1), row),
            pl.BlockSpec((blk, 1), row),
            pl.BlockSpec((3 * FEAT, 128), rep),
            pl.BlockSpec((1, 128), rep),
            pl.BlockSpec((128, 64), rep),
            pl.BlockSpec((1, 64), rep),
            pl.BlockSpec((64, 128), rep),
            pl.BlockSpec((1, 128), rep),
        ],
        out_specs=pl.BlockSpec((blk, 128), row),
        out_shape=jax.ShapeDtypeStruct((B, 128), jnp.float32),
    )(id_emb, loc_emb, age_emb, fid, floc, fage, W1, b1.reshape(1, -1),
      W2, b2.reshape(1, -1), W3, b3.reshape(1, -1))


def kernel(user_ids, user_locations, user_ages, id_table, loc_table, age_table,
           W1, b1, W2, b2, W3, b3):
    info = plsc.get_sparse_core_info()
    nw = info.num_cores * info.num_subcores
    b_per_w = B // nw
    n_chunks = b_per_w // CHUNK

    ids = user_ids.astype(jnp.int32)
    locs = user_locations.astype(jnp.int32)
    ages = user_ages.astype(jnp.int32)

    def pair_idx(r, lblk):
        q = lax.rem(r, lblk)
        half = lblk // 2
        p = (r // lblk) * half + lax.rem(q, half)
        flag = (q >= half).astype(jnp.int32)
        return p, flag

    loc_pairs = _pairs(loc_table.T, 100000, 8192)
    age_pairs = _pairs(age_table.T, 1000, 1000)
    id_pairs = _pairs(id_table.T, 1000000, 8192)

    p_id, f_id = pair_idx(ids, 8192)
    p_loc, f_loc = pair_idx(locs, 8192)
    p_age, f_age = pair_idx(ages, 1000)

    loc_emb, age_emb = _gather_n([
        (p_loc.reshape(nw, n_chunks, CHUNK), loc_pairs),
        (p_age.reshape(nw, n_chunks, CHUNK), age_pairs),
    ])
    (id_emb,) = _gather_n([
        (p_id.reshape(nw, n_chunks, CHUNK), id_pairs),
    ])
    return _mlp(id_emb, loc_emb, age_emb,
                f_id.reshape(B, 1), f_loc.reshape(B, 1), f_age.reshape(B, 1),
                W1, b1, W2, b2, W3, b3)


# speedup vs baseline: 1.1971x; 1.0958x over previous
import functools
import math

import jax
import jax.numpy as jnp
from jax import lax
from jax.experimental import pallas as pl
from jax.experimental.pallas import tpu as pltpu
from jax.experimental.pallas import tpu_sc as plsc

B = 16384
FEAT = 64
CHUNK = 128


def _pairs_body(a_ref, out_ref, *, vocab):
    a = a_ref[...]
    lane = (lax.broadcasted_iota(jnp.int32, a.shape, 1)
            + pl.program_id(0) * a.shape[1])
    a = jnp.where(lane < vocab, a, 0.0)
    p = a.shape[1] // 2
    a2 = jnp.concatenate([a[:, :p], a[:, p:]], axis=0).astype(jnp.bfloat16)
    ii = lax.broadcasted_iota(jnp.int32, (2 * FEAT, 2 * FEAT), 0)
    jj = lax.broadcasted_iota(jnp.int32, (2 * FEAT, 2 * FEAT), 1)
    eye = jnp.where(ii == jj, 1.0, 0.0).astype(jnp.bfloat16)
    out_ref[...] = lax.dot_general(a2, eye, (((0,), (0,)), ((), ())),
                                   preferred_element_type=jnp.float32)


def _pairs(tab_t, vocab, lblk):
    grid = ((vocab + lblk - 1) // lblk,)
    half = grid[0] * (lblk // 2)
    return pl.pallas_call(
        functools.partial(_pairs_body, vocab=vocab),
        grid=grid,
        in_specs=[pl.BlockSpec((FEAT, lblk), lambda i: (0, i))],
        out_specs=pl.BlockSpec((lblk // 2, 2 * FEAT), lambda i: (i, 0)),
        out_shape=jax.ShapeDtypeStruct((half, 2 * FEAT), jnp.float32),
    )(tab_t)


def _gather_n(idx_tabs):
    n = len(idx_tabs)
    info = plsc.get_sparse_core_info()
    nw = info.num_cores * info.num_subcores
    b_per_w = B // nw
    n_chunks = b_per_w // CHUNK

    mesh = plsc.VectorSubcoreMesh(core_axis_name="c", subcore_axis_name="s")

    @functools.partial(
        pl.kernel,
        mesh=mesh,
        out_type=[jax.ShapeDtypeStruct((B, 2 * FEAT), jnp.float32)] * n,
        scratch_types=(
            [pltpu.VMEM((n_chunks, CHUNK), jnp.int32)] * n
            + [pltpu.VMEM((2, CHUNK, 2 * FEAT), jnp.float32)] * n
            + [pltpu.SemaphoreType.DMA] * 2
        ),
    )
    def gather_k(*refs):
        idx_r = refs[:n]
        tabs = refs[n:2 * n]
        outs = refs[2 * n:3 * n]
        idxs = refs[3 * n:4 * n]
        rows = refs[4 * n:5 * n]
        gsem, wsem = refs[5 * n:]
        wid = lax.axis_index("s") * info.num_cores + lax.axis_index("c")
        base = wid * b_per_w
        for t in range(n):
            pltpu.sync_copy(idx_r[t].at[wid], idxs[t])
        gathers = [[None] * n for _ in range(n_chunks)]
        writes = [[None] * n for _ in range(n_chunks)]
        for j in range(n_chunks):
            b = j % 2
            if j >= 2:
                for t in range(n):
                    writes[j - 2][t].wait()
            for t in range(n):
                gathers[j][t] = pltpu.async_copy(
                    tabs[t].at[idxs[t].at[j]], rows[t].at[b], gsem)
            for t in range(n):
                gathers[j][t].wait()
            dst = pl.ds(base + j * CHUNK, CHUNK)
            for t in range(n):
                writes[j][t] = pltpu.async_copy(rows[t].at[b], outs[t].at[dst], wsem)
        for j in range(n_chunks - 2, n_chunks):
            for t in range(n):
                writes[j][t].wait()

    args = [it[0] for it in idx_tabs] + [it[1] for it in idx_tabs]
    res = gather_k(*args)
    return list(res) if isinstance(res, (list, tuple)) else [res]


_INV_SQRT2 = 1.0 / math.sqrt(2.0)


def _gelu(x):
    return 0.5 * x * (1.0 + lax.erf(x * _INV_SQRT2))


def _ln(x, eps=1e-5):
    mu = jnp.mean(x, axis=-1, keepdims=True)
    var = jnp.mean((x - mu) * (x - mu), axis=-1, keepdims=True)
    return (x - mu) * lax.rsqrt(var + eps)


def _half(buf, flag_ref):
    p = flag_ref[...] != 0
    return jnp.where(p, buf[:, FEAT:2 * FEAT], buf[:, 0:FEAT])


def _mlp_body(id_ref, loc_ref, age_ref, fid_ref, floc_ref, fage_ref,
              w1_ref, b1_ref, w2_ref, b2_ref, w3_ref, b3_ref, out_ref):
    w1 = w1_ref[...]
    id_emb = _half(id_ref[...], fid_ref)
    loc_emb = _half(loc_ref[...], floc_ref)
    age_emb = _half(age_ref[...], fage_ref)
    h = (
        jnp.dot(id_emb, w1[0:FEAT], preferred_element_type=jnp.float32)
        + jnp.dot(loc_emb, w1[FEAT:2 * FEAT], preferred_element_type=jnp.float32)
        + jnp.dot(age_emb, w1[2 * FEAT:3 * FEAT], preferred_element_type=jnp.float32)
        + b1_ref[...]
    )
    h = _gelu(_ln(h))
    h = jnp.dot(h, w2_ref[...], preferred_element_type=jnp.float32) + b2_ref[...]
    h = _gelu(_ln(h))
    h = jnp.dot(h, w3_ref[...], preferred_element_type=jnp.float32) + b3_ref[...]
    out_ref[...] = _gelu(h)


def _mlp(id_emb, loc_emb, age_emb, fid, floc, fage, W1, b1, W2, b2, W3, b3,
         blk=2048):
    grid = (B // blk,)
    rep = lambda i: (0, 0)
    row = lambda i: (i, 0)
    return pl.pallas_call(
        _mlp_body,
        grid=grid,
        in_specs=[
            pl.BlockSpec((blk, 2 * FEAT), row),
            pl.BlockSpec((blk, 2 * FEAT), row),
            pl.BlockSpec((blk, 2 * FEAT), row),
            pl.BlockSpec((blk, 1), row),
            pl.BlockSpec((blk, 1), row),
            pl.BlockSpec((blk, 1), row),
            pl.BlockSpec((3 * FEAT, 128), rep),
            pl.BlockSpec((1, 128), rep),
            pl.BlockSpec((128, 64), rep),
            pl.BlockSpec((1, 64), rep),
            pl.BlockSpec((64, 128), rep),
            pl.BlockSpec((1, 128), rep),
        ],
        out_specs=pl.BlockSpec((blk, 128), row),
        out_shape=jax.ShapeDtypeStruct((B, 128), jnp.float32),
    )(id_emb, loc_emb, age_emb, fid, floc, fage, W1, b1.reshape(1, -1),
      W2, b2.reshape(1, -1), W3, b3.reshape(1, -1))


def kernel(user_ids, user_locations, user_ages, id_table, loc_table, age_table,
           W1, b1, W2, b2, W3, b3):
    info = plsc.get_sparse_core_info()
    nw = info.num_cores * info.num_subcores
    b_per_w = B // nw
    n_chunks = b_per_w // CHUNK

    ids = user_ids.astype(jnp.int32)
    locs = user_locations.astype(jnp.int32)
    ages = user_ages.astype(jnp.int32)

    def pair_idx(r, lblk):
        q = lax.rem(r, lblk)
        half = lblk // 2
        p = (r // lblk) * half + lax.rem(q, half)
        flag = (q >= half).astype(jnp.int32)
        return p, flag

    loc_pairs = _pairs(loc_table.T, 100000, 16384)
    age_pairs = _pairs(age_table.T, 1000, 1000)
    id_pairs = _pairs(id_table.T, 1000000, 16384)

    p_id, f_id = pair_idx(ids, 16384)
    p_loc, f_loc = pair_idx(locs, 16384)
    p_age, f_age = pair_idx(ages, 1000)

    loc_emb, age_emb = _gather_n([
        (p_loc.reshape(nw, n_chunks, CHUNK), loc_pairs),
        (p_age.reshape(nw, n_chunks, CHUNK), age_pairs),
    ])
    (id_emb,) = _gather_n([
        (p_id.reshape(nw, n_chunks, CHUNK), id_pairs),
    ])
    return _mlp(id_emb, loc_emb, age_emb,
                f_id.reshape(B, 1), f_loc.reshape(B, 1), f_age.reshape(B, 1),
                W1, b1, W2, b2, W3, b3)


# speedup vs baseline: 1.2079x; 1.0090x over previous
import functools
import math

import jax
import jax.numpy as jnp
from jax import lax
from jax.experimental import pallas as pl
from jax.experimental.pallas import tpu as pltpu
from jax.experimental.pallas import tpu_sc as plsc

B = 16384
FEAT = 64
CHUNK = 128


def _pairs_body(a_ref, out_ref, *, vocab):
    a = a_ref[...]
    lane = (lax.broadcasted_iota(jnp.int32, a.shape, 1)
            + pl.program_id(0) * a.shape[1])
    a = jnp.where(lane < vocab, a, 0.0)
    p = a.shape[1] // 2
    a2 = jnp.concatenate([a[:, :p], a[:, p:]], axis=0).astype(jnp.bfloat16)
    ii = lax.broadcasted_iota(jnp.int32, (2 * FEAT, 2 * FEAT), 0)
    jj = lax.broadcasted_iota(jnp.int32, (2 * FEAT, 2 * FEAT), 1)
    eye = jnp.where(ii == jj, 1.0, 0.0).astype(jnp.bfloat16)
    out_ref[...] = lax.dot_general(a2, eye, (((0,), (0,)), ((), ())),
                                   preferred_element_type=jnp.float32)


def _pairs(tab_t, vocab, lblk):
    grid = ((vocab + lblk - 1) // lblk,)
    half = grid[0] * (lblk // 2)
    return pl.pallas_call(
        functools.partial(_pairs_body, vocab=vocab),
        grid=grid,
        in_specs=[pl.BlockSpec((FEAT, lblk), lambda i: (0, i))],
        out_specs=pl.BlockSpec((lblk // 2, 2 * FEAT), lambda i: (i, 0)),
        out_shape=jax.ShapeDtypeStruct((half, 2 * FEAT), jnp.float32),
    )(tab_t)


def _gather_n(idx_tabs):
    n = len(idx_tabs)
    info = plsc.get_sparse_core_info()
    nw = info.num_cores * info.num_subcores
    b_per_w = B // nw
    n_chunks = b_per_w // CHUNK

    mesh = plsc.VectorSubcoreMesh(core_axis_name="c", subcore_axis_name="s")

    @functools.partial(
        pl.kernel,
        mesh=mesh,
        out_type=[jax.ShapeDtypeStruct((B, 2 * FEAT), jnp.float32)] * n,
        scratch_types=(
            [pltpu.VMEM((n_chunks, CHUNK), jnp.int32)] * n
            + [pltpu.VMEM((2, CHUNK, 2 * FEAT), jnp.float32)] * n
            + [pltpu.SemaphoreType.DMA] * 2
        ),
    )
    def gather_k(*refs):
        idx_r = refs[:n]
        tabs = refs[n:2 * n]
        outs = refs[2 * n:3 * n]
        idxs = refs[3 * n:4 * n]
        rows = refs[4 * n:5 * n]
        gsem, wsem = refs[5 * n:]
        wid = lax.axis_index("s") * info.num_cores + lax.axis_index("c")
        base = wid * b_per_w
        for t in range(n):
            pltpu.sync_copy(idx_r[t].at[wid], idxs[t])
        gathers = [[None] * n for _ in range(n_chunks)]
        writes = [[None] * n for _ in range(n_chunks)]
        for j in range(n_chunks):
            b = j % 2
            if j >= 2:
                for t in range(n):
                    writes[j - 2][t].wait()
            for t in range(n):
                gathers[j][t] = pltpu.async_copy(
                    tabs[t].at[idxs[t].at[j]], rows[t].at[b], gsem)
            for t in range(n):
                gathers[j][t].wait()
            dst = pl.ds(base + j * CHUNK, CHUNK)
            for t in range(n):
                writes[j][t] = pltpu.async_copy(rows[t].at[b], outs[t].at[dst], wsem)
        for j in range(n_chunks - 2, n_chunks):
            for t in range(n):
                writes[j][t].wait()

    args = [it[0] for it in idx_tabs] + [it[1] for it in idx_tabs]
    res = gather_k(*args)
    return list(res) if isinstance(res, (list, tuple)) else [res]


_INV_SQRT2 = 1.0 / math.sqrt(2.0)


def _gelu(x):
    return 0.5 * x * (1.0 + lax.erf(x * _INV_SQRT2))


def _ln(x, eps=1e-5):
    mu = jnp.mean(x, axis=-1, keepdims=True)
    var = jnp.mean((x - mu) * (x - mu), axis=-1, keepdims=True)
    return (x - mu) * lax.rsqrt(var + eps)


def _half(buf, flag_ref):
    p = flag_ref[...] != 0
    return jnp.where(p, buf[:, FEAT:2 * FEAT], buf[:, 0:FEAT])


def _mlp_body(id_ref, loc_ref, age_ref, fid_ref, floc_ref, fage_ref,
              w1_ref, b1_ref, w2_ref, b2_ref, w3_ref, b3_ref, out_ref):
    w1 = w1_ref[...]
    id_emb = _half(id_ref[...], fid_ref)
    loc_emb = _half(loc_ref[...], floc_ref)
    age_emb = _half(age_ref[...], fage_ref)
    h = (
        jnp.dot(id_emb, w1[0:FEAT], preferred_element_type=jnp.float32)
        + jnp.dot(loc_emb, w1[FEAT:2 * FEAT], preferred_element_type=jnp.float32)
        + jnp.dot(age_emb, w1[2 * FEAT:3 * FEAT], preferred_element_type=jnp.float32)
        + b1_ref[...]
    )
    h = _gelu(_ln(h))
    h = jnp.dot(h, w2_ref[...], preferred_element_type=jnp.float32) + b2_ref[...]
    h = _gelu(_ln(h))
    h = jnp.dot(h, w3_ref[...], preferred_element_type=jnp.float32) + b3_ref[...]
    out_ref[...] = _gelu(h)


def _mlp(id_emb, loc_emb, age_emb, fid, floc, fage, W1, b1, W2, b2, W3, b3,
         blk=2048):
    grid = (B // blk,)
    rep = lambda i: (0, 0)
    row = lambda i: (i, 0)
    return pl.pallas_call(
        _mlp_body,
        grid=grid,
        in_specs=[
            pl.BlockSpec((blk, 2 * FEAT), row),
            pl.BlockSpec((blk, 2 * FEAT), row),
            pl.BlockSpec((blk, 2 * FEAT), row),
            pl.BlockSpec((blk, 1), row),
            pl.BlockSpec((blk, 1), row),
            pl.BlockSpec((blk, 1), row),
            pl.BlockSpec((3 * FEAT, 128), rep),
            pl.BlockSpec((1, 128), rep),
            pl.BlockSpec((128, 64), rep),
            pl.BlockSpec((1, 64), rep),
            pl.BlockSpec((64, 128), rep),
            pl.BlockSpec((1, 128), rep),
        ],
        out_specs=pl.BlockSpec((blk, 128), row),
        out_shape=jax.ShapeDtypeStruct((B, 128), jnp.float32),
    )(id_emb, loc_emb, age_emb, fid, floc, fage, W1, b1.reshape(1, -1),
      W2, b2.reshape(1, -1), W3, b3.reshape(1, -1))


def kernel(user_ids, user_locations, user_ages, id_table, loc_table, age_table,
           W1, b1, W2, b2, W3, b3):
    info = plsc.get_sparse_core_info()
    nw = info.num_cores * info.num_subcores
    b_per_w = B // nw
    n_chunks = b_per_w // CHUNK

    ids = user_ids.astype(jnp.int32)
    locs = user_locations.astype(jnp.int32)
    ages = user_ages.astype(jnp.int32)

    def pair_idx(r, lblk):
        q = lax.rem(r, lblk)
        half = lblk // 2
        p = (r // lblk) * half + lax.rem(q, half)
        flag = (q >= half).astype(jnp.int32)
        return p, flag

    loc_pairs = _pairs(loc_table.T, 100000, 32768)
    age_pairs = _pairs(age_table.T, 1000, 1000)
    id_pairs = _pairs(id_table.T, 1000000, 32768)

    p_id, f_id = pair_idx(ids, 32768)
    p_loc, f_loc = pair_idx(locs, 32768)
    p_age, f_age = pair_idx(ages, 1000)

    loc_emb, age_emb = _gather_n([
        (p_loc.reshape(nw, n_chunks, CHUNK), loc_pairs),
        (p_age.reshape(nw, n_chunks, CHUNK), age_pairs),
    ])
    (id_emb,) = _gather_n([
        (p_id.reshape(nw, n_chunks, CHUNK), id_pairs),
    ])
    return _mlp(id_emb, loc_emb, age_emb,
                f_id.reshape(B, 1), f_loc.reshape(B, 1), f_age.reshape(B, 1),
                W1, b1, W2, b2, W3, b3)
